# Initial kernel scaffold; baseline (speedup 1.0000x reference)
import jax
import jax.numpy as jnp
from jax.experimental import pallas as pl


def kernel(input, weight):
    raise NotImplementedError("write your pallas kernel here")


# speedup vs baseline: 1.5471x; 1.5471x over previous
import functools

import jax
import jax.numpy as jnp
from jax import lax
from jax.experimental import pallas as pl
from jax.experimental.pallas import tpu as pltpu
from jax.experimental.pallas import tpu_sc as plsc

BATCH = 16384
FIELDS = 26
EMBED = 32
TOTAL = BATCH * FIELDS

NC = 2
NS = 16
NW = NC * NS
PER_W = TOTAL // NW
CHUNK = 1024
NCHUNK = PER_W // CHUNK


def _emb_body(idx_hbm, table_hbm, out_hbm, idx_v, rows_v, sem):
    wid = lax.axis_index("s") * NC + lax.axis_index("c")
    base = wid * PER_W

    def chunk_step(c, carry):
        off = base + c * CHUNK
        pltpu.sync_copy(idx_hbm.at[pl.ds(off, CHUNK)], idx_v)
        pltpu.async_copy(table_hbm.at[idx_v], rows_v, sem).wait()
        pltpu.sync_copy(rows_v, out_hbm.at[pl.ds(off, CHUNK)])
        return carry

    lax.fori_loop(0, NCHUNK, chunk_step, 0)


@jax.jit
def _emb(idx_flat, weight):
    mesh = plsc.VectorSubcoreMesh(core_axis_name="c", subcore_axis_name="s")
    run = pl.kernel(
        _emb_body,
        out_type=jax.ShapeDtypeStruct((TOTAL, EMBED), jnp.float32),
        mesh=mesh,
        scratch_types=[
            pltpu.VMEM((CHUNK,), jnp.int32),
            pltpu.VMEM((CHUNK, EMBED), jnp.float32),
            pltpu.SemaphoreType.DMA,
        ],
        compiler_params=pltpu.CompilerParams(use_tc_tiling_on_sc=False),
    )
    return run(idx_flat, weight)


def kernel(input, weight):
    idx_flat = input.reshape(TOTAL).astype(jnp.int32)
    out = _emb(idx_flat, weight)
    return out.reshape(BATCH, FIELDS, EMBED)


# speedup vs baseline: 1.5762x; 1.0188x over previous
import functools

import jax
import jax.numpy as jnp
from jax import lax
from jax.experimental import pallas as pl
from jax.experimental.pallas import tpu as pltpu
from jax.experimental.pallas import tpu_sc as plsc

BATCH = 16384
FIELDS = 26
EMBED = 32
TOTAL = BATCH * FIELDS

NC = 2
NS = 16
NW = NC * NS
PER_W = TOTAL // NW
CHUNK = 1664
NCHUNK = PER_W // CHUNK


def _emb_body(idx_hbm, table_hbm, out_hbm, idx_v, rows_v,
              gsem0, gsem1, wsem0, wsem1):
    wid = lax.axis_index("s") * NC + lax.axis_index("c")
    base = wid * PER_W
    gsems = (gsem0, gsem1)
    wsems = (wsem0, wsem1)

    pltpu.sync_copy(idx_hbm.at[pl.ds(base, CHUNK)], idx_v.at[0])
    gathers = [pltpu.async_copy(table_hbm.at[idx_v.at[0]], rows_v.at[0],
                                gsems[0])]
    writes = [None, None]
    for c in range(NCHUNK):
        b = c % 2
        nb = (c + 1) % 2
        if c + 1 < NCHUNK:
            off = base + (c + 1) * CHUNK
            pltpu.sync_copy(idx_hbm.at[pl.ds(off, CHUNK)], idx_v.at[nb])
            if writes[nb] is not None:
                writes[nb].wait()
            gathers.append(pltpu.async_copy(
                table_hbm.at[idx_v.at[nb]], rows_v.at[nb], gsems[nb]))
        gathers[c].wait()
        writes[b] = pltpu.async_copy(
            rows_v.at[b], out_hbm.at[pl.ds(base + c * CHUNK, CHUNK)],
            wsems[b])
    for w in writes:
        if w is not None:
            w.wait()


@jax.jit
def _emb(idx_flat, weight):
    mesh = plsc.VectorSubcoreMesh(core_axis_name="c", subcore_axis_name="s")
    run = pl.kernel(
        _emb_body,
        out_type=jax.ShapeDtypeStruct((TOTAL, EMBED), jnp.float32),
        mesh=mesh,
        scratch_types=[
            pltpu.VMEM((2, CHUNK), jnp.int32),
            pltpu.VMEM((2, CHUNK, EMBED), jnp.float32),
            pltpu.SemaphoreType.DMA,
            pltpu.SemaphoreType.DMA,
            pltpu.SemaphoreType.DMA,
            pltpu.SemaphoreType.DMA,
        ],
        compiler_params=pltpu.CompilerParams(use_tc_tiling_on_sc=False),
    )
    return run(idx_flat, weight)


def kernel(input, weight):
    idx_flat = input.reshape(TOTAL).astype(jnp.int32)
    out = _emb(idx_flat, weight)
    return out.reshape(BATCH, FIELDS, EMBED)
